# baseline (speedup 1.0000x reference)
import functools

import jax
import jax.numpy as jnp
from jax import lax
from jax.experimental import pallas as pl
from jax.experimental.pallas import tpu as pltpu
from jax.experimental.pallas import tpu_sc as plsc

_BN_EPS = 1e-5


def _stats_kernel(x_ref, w1_ref, out_ref):
    @pl.when(pl.program_id(0) == 0)
    def _():
        out_ref[...] = jnp.zeros_like(out_ref)

    x = x_ref[...]
    a = jnp.dot(x, w1_ref[...], preferred_element_type=jnp.float32)
    hid = a.shape[1]
    sh = jnp.sum(a, axis=0)
    sh2 = jnp.sum(a * a, axis=0)
    sx2 = jnp.sum(x * x)
    pad = jnp.zeros((128 - hid,), jnp.float32)
    rows = lax.broadcasted_iota(jnp.int32, (8, 128), 0)
    cols = lax.broadcasted_iota(jnp.int32, (8, 128), 1)
    upd = (jnp.where(rows == 0, jnp.concatenate([sh, pad])[None, :], 0.0)
           + jnp.where(rows == 1, jnp.concatenate([sh2, pad])[None, :], 0.0)
           + jnp.where((rows == 2) & (cols == 0), sx2, 0.0))
    out_ref[...] += upd


def _main_kernel(x_ref, w1_ref, w2_ref, cb_ref, b1e_ref, g1e_ref,
                 be1e_ref, b2e_ref, stats_ref, topics_ref, s_ref, st_ref,
                 aff_scr, cc_scr, *, n_rows):
    i = pl.program_id(0)
    hid = w1_ref.shape[1]

    @pl.when(i == 0)
    def _():
        cb0 = cb_ref[...]
        cc = jnp.sum(cb0 * cb0, axis=1)
        cc_scr[...] = jnp.broadcast_to(cc[:, None], cc_scr.shape)
        inv_n = 1.0 / n_rows
        mean_a = stats_ref[0:1, :hid] * inv_n
        var = stats_ref[1:2, :hid] * inv_n - mean_a * mean_a
        mean = mean_a + b1e_ref[...][None, :]
        scale = g1e_ref[...][None, :] * lax.rsqrt(var + _BN_EPS)
        shift = be1e_ref[...][None, :] - mean * scale
        rows8 = lax.broadcasted_iota(jnp.int32, (8, 128), 0)
        cols8 = lax.broadcasted_iota(jnp.int32, (8, 128), 1)
        pad = jnp.zeros((1, 128 - hid), jnp.float32)
        aff_scr[...] = (
            jnp.where(rows8 == 0, jnp.concatenate([scale, pad], axis=1), 0.0)
            + jnp.where(rows8 == 1, jnp.concatenate([shift, pad], axis=1),
                        0.0))
        s_ref[...] = jnp.zeros_like(s_ref)
        st_ref[...] = jnp.where((rows8 == 2) & (cols8 == 0),
                                stats_ref[...], 0.0)

    x = x_ref[...]
    b = x.shape[0]
    aff = aff_scr[...]
    h = jnp.dot(x, w1_ref[...], preferred_element_type=jnp.float32)
    h = h + b1e_ref[...][None, :]
    h = h * aff[0:1, :hid] + aff[1:2, :hid]
    h = jnp.maximum(h, 0.0)
    z = jnp.dot(h, w2_ref[...], preferred_element_type=jnp.float32)
    z = z + b2e_ref[...][None, :]

    cbm2 = cb_ref[...] * -2.0
    k = cbm2.shape[0]
    nsub = 2
    b2 = b // nsub
    zl = 0.0
    s_acc = None
    for s in range(nsub):
        xs = x[s * b2:(s + 1) * b2, :]
        zs = z[s * b2:(s + 1) * b2, :]
        zz = jnp.sum(zs * zs, axis=1)
        zcm2 = lax.dot_general(cbm2, zs, (((1,), (1,)), ((), ())),
                               preferred_element_type=jnp.float32)
        d = (cc_scr[:, 0:1] + zz) + zcm2

        mind = jnp.min(d, axis=0)
        ismin = d == mind[None, :]
        rowid = lax.broadcasted_iota(jnp.int32, (k, b2), 0)
        idx = jnp.min(jnp.where(ismin, rowid, k), axis=0)
        topics_ref[pl.ds(s * b2, b2)] = idx

        ds = lax.dot_general(ismin.astype(jnp.bfloat16),
                             xs.astype(jnp.bfloat16),
                             (((1,), (0,)), ((), ())),
                             preferred_element_type=jnp.float32)
        s_acc = ds if s_acc is None else s_acc + ds
        zl = zl + jnp.sum(mind)

    s_ref[...] += s_acc
    rows = lax.broadcasted_iota(jnp.int32, (8, 128), 0)
    cols = lax.broadcasted_iota(jnp.int32, (8, 128), 1)
    st_ref[...] += jnp.where((rows == 0) & (cols == 0), zl, 0.0)


def _hist_body(topics_hbm, out_hbm, idx_v, acc_v, red_v):
    nc = 2
    wid = lax.axis_index("s") * nc + lax.axis_index("c")
    shard = 2048
    kk = 1024
    base = wid * shard
    pltpu.sync_copy(topics_hbm.at[pl.ds(base, shard)], idx_v)

    zeros16 = jnp.zeros((16,), jnp.float32)

    def zbody(m, _):
        acc_v[pl.ds(m * 16, 16)] = zeros16
        return _
    lax.fori_loop(0, (16 * kk) // 16, zbody, None)

    lane = lax.broadcasted_iota(jnp.int32, (16,), 0)
    bank = lane * kk
    ones16 = jnp.ones((16,), jnp.float32)

    def sbody(j, _):
        t = idx_v[pl.ds(j * 16, 16)]
        plsc.addupdate_scatter(acc_v, [bank + t], ones16)
        return _
    lax.fori_loop(0, shard // 16, sbody, None)

    def rbody(i, _):
        s = zeros16
        for j in range(16):
            s = s + acc_v[pl.ds(j * kk + i * 16, 16)]
        red_v[pl.ds(i * 16, 16)] = s
        return _
    lax.fori_loop(0, kk // 16, rbody, None)

    pltpu.sync_copy(red_v, out_hbm.at[wid])


def _histogram_sc(topics, n, k):
    mesh = plsc.VectorSubcoreMesh(core_axis_name="c", subcore_axis_name="s")
    return pl.kernel(
        _hist_body,
        mesh=mesh,
        out_type=jax.ShapeDtypeStruct((32, k), jnp.float32),
        scratch_types=[pltpu.VMEM((n // 32,), jnp.int32),
                       pltpu.VMEM((16 * k,), jnp.float32),
                       pltpu.VMEM((k,), jnp.float32)],
        compiler_params=pltpu.CompilerParams(needs_layout_passes=False),
    )(topics)


def _combine_kernel(cb_ref, w1d_ref, w2d_ref, b1d_ref, g1d_ref, be1d_ref,
                    b2d_ref, cnt_ref, s_ref, st_ref, out_ref, *, n_rows):
    cb = cb_ref[...]
    hd = jnp.dot(cb, w1d_ref[...], preferred_element_type=jnp.float32)
    hd = hd + b1d_ref[...][None, :]
    counts = jnp.sum(cnt_ref[...], axis=0, keepdims=True)
    inv_n = 1.0 / n_rows
    mu = jnp.dot(counts, hd, preferred_element_type=jnp.float32) * inv_n
    ex2 = jnp.dot(counts, hd * hd, preferred_element_type=jnp.float32) * inv_n
    var = ex2 - mu * mu
    scale = g1d_ref[...][None, :] / jnp.sqrt(var + _BN_EPS)
    shift = be1d_ref[...][None, :] - mu * scale
    a = jnp.maximum(hd * scale + shift, 0.0)
    dec = jnp.dot(a, w2d_ref[...], preferred_element_type=jnp.float32)
    dec = dec + b2d_ref[...][None, :]

    d2 = jnp.sum(dec * dec, axis=1)
    cterm = jnp.sum(counts[0, :] * d2)
    cross = jnp.sum(dec * s_ref[...])

    r1 = lax.broadcasted_iota(jnp.int32, (8, 128), 0)
    c1 = lax.broadcasted_iota(jnp.int32, (8, 128), 1)
    stats = st_ref[...]
    sumx2 = jnp.sum(jnp.where((r1 == 2) & (c1 == 0), stats, 0.0))
    zloss = jnp.sum(jnp.where((r1 == 0) & (c1 == 0), stats, 0.0))

    recon = sumx2 + cterm - 2.0 * cross
    loss = 2.0 * zloss + jnp.sqrt(recon)
    out_ref[...] = jnp.full((8, 128), loss, dtype=jnp.float32)


def kernel(X, W1e, b1e, g1e, be1e, W2e, b2e, codebook,
           W1d, b1d, g1d, be1d, W2d, b2d):
    N, d_in = X.shape
    hid = W1e.shape[1]
    code = W2e.shape[1]
    K = codebook.shape[0]
    BN = 4096
    grid = N // BN

    stats1 = pl.pallas_call(
        _stats_kernel,
        grid=(grid,),
        in_specs=[pl.BlockSpec((BN, d_in), lambda i: (i, 0)),
                  pl.BlockSpec((d_in, hid), lambda i: (0, 0))],
        out_specs=pl.BlockSpec((8, 128), lambda i: (0, 0)),
        out_shape=jax.ShapeDtypeStruct((8, 128), jnp.float32),
    )(X, W1e)

    topics, S, st2 = pl.pallas_call(
        functools.partial(_main_kernel, n_rows=float(N)),
        grid=(grid,),
        in_specs=[pl.BlockSpec((BN, d_in), lambda i: (i, 0)),
                  pl.BlockSpec((d_in, hid), lambda i: (0, 0)),
                  pl.BlockSpec((hid, code), lambda i: (0, 0)),
                  pl.BlockSpec((K, code), lambda i: (0, 0)),
                  pl.BlockSpec((hid,), lambda i: (0,)),
                  pl.BlockSpec((hid,), lambda i: (0,)),
                  pl.BlockSpec((hid,), lambda i: (0,)),
                  pl.BlockSpec((code,), lambda i: (0,)),
                  pl.BlockSpec((8, 128), lambda i: (0, 0))],
        out_specs=[pl.BlockSpec((BN,), lambda i: (i,)),
                   pl.BlockSpec((K, d_in), lambda i: (0, 0)),
                   pl.BlockSpec((8, 128), lambda i: (0, 0))],
        out_shape=[jax.ShapeDtypeStruct((N,), jnp.int32),
                   jax.ShapeDtypeStruct((K, d_in), jnp.float32),
                   jax.ShapeDtypeStruct((8, 128), jnp.float32)],
        scratch_shapes=[pltpu.VMEM((8, 128), jnp.float32),
                        pltpu.VMEM((K, 8), jnp.float32)],
    )(X, W1e, W2e, codebook, b1e, g1e, be1e, b2e, stats1)

    counts_part = _histogram_sc(topics, N, K)

    loss_tile = pl.pallas_call(
        functools.partial(_combine_kernel, n_rows=float(N)),
        grid=(1,),
        in_specs=[pl.BlockSpec((K, code), lambda i: (0, 0)),
                  pl.BlockSpec((code, hid), lambda i: (0, 0)),
                  pl.BlockSpec((hid, d_in), lambda i: (0, 0)),
                  pl.BlockSpec((hid,), lambda i: (0,)),
                  pl.BlockSpec((hid,), lambda i: (0,)),
                  pl.BlockSpec((hid,), lambda i: (0,)),
                  pl.BlockSpec((d_in,), lambda i: (0,)),
                  pl.BlockSpec((32, K), lambda i: (0, 0)),
                  pl.BlockSpec((K, d_in), lambda i: (0, 0)),
                  pl.BlockSpec((8, 128), lambda i: (0, 0))],
        out_specs=pl.BlockSpec((8, 128), lambda i: (0, 0)),
        out_shape=jax.ShapeDtypeStruct((8, 128), jnp.float32),
    )(codebook, W1d, W2d, b1d, g1d, be1d, b2d, counts_part, S, st2)

    return (loss_tile[0, 0], topics)


# speedup vs baseline: 1.0446x; 1.0446x over previous
import functools

import jax
import jax.numpy as jnp
from jax import lax
from jax.experimental import pallas as pl
from jax.experimental.pallas import tpu as pltpu
from jax.experimental.pallas import tpu_sc as plsc

_BN_EPS = 1e-5


def _stats_kernel(x_ref, w1_ref, out_ref):
    @pl.when(pl.program_id(0) == 0)
    def _():
        out_ref[...] = jnp.zeros_like(out_ref)

    x = x_ref[...]
    a = jnp.dot(x, w1_ref[...], preferred_element_type=jnp.float32)
    hid = a.shape[1]
    sh = jnp.sum(a, axis=0)
    sh2 = jnp.sum(a * a, axis=0)
    sx2 = jnp.sum(x * x)
    pad = jnp.zeros((128 - hid,), jnp.float32)
    rows = lax.broadcasted_iota(jnp.int32, (8, 128), 0)
    cols = lax.broadcasted_iota(jnp.int32, (8, 128), 1)
    upd = (jnp.where(rows == 0, jnp.concatenate([sh, pad])[None, :], 0.0)
           + jnp.where(rows == 1, jnp.concatenate([sh2, pad])[None, :], 0.0)
           + jnp.where((rows == 2) & (cols == 0), sx2, 0.0))
    out_ref[...] += upd


def _main_kernel(x_ref, w1_ref, w2_ref, cb_ref, b1e_ref, g1e_ref,
                 be1e_ref, b2e_ref, stats_ref, topics_ref, s_ref, st_ref,
                 aff_scr, cc_scr, *, n_rows):
    i = pl.program_id(0)
    hid = w1_ref.shape[1]

    @pl.when(i == 0)
    def _():
        cb0 = cb_ref[...]
        cc = jnp.sum(cb0 * cb0, axis=1)
        cc_scr[...] = jnp.broadcast_to(cc[:, None], cc_scr.shape)
        inv_n = 1.0 / n_rows
        mean_a = stats_ref[0:1, :hid] * inv_n
        var = stats_ref[1:2, :hid] * inv_n - mean_a * mean_a
        mean = mean_a + b1e_ref[...][None, :]
        scale = g1e_ref[...][None, :] * lax.rsqrt(var + _BN_EPS)
        shift = be1e_ref[...][None, :] - mean * scale
        rows8 = lax.broadcasted_iota(jnp.int32, (8, 128), 0)
        cols8 = lax.broadcasted_iota(jnp.int32, (8, 128), 1)
        pad = jnp.zeros((1, 128 - hid), jnp.float32)
        aff_scr[...] = (
            jnp.where(rows8 == 0, jnp.concatenate([scale, pad], axis=1), 0.0)
            + jnp.where(rows8 == 1, jnp.concatenate([shift, pad], axis=1),
                        0.0))
        s_ref[...] = jnp.zeros_like(s_ref)
        st_ref[...] = jnp.where((rows8 == 2) & (cols8 == 0),
                                stats_ref[...], 0.0)

    x = x_ref[...]
    b = x.shape[0]
    aff = aff_scr[...]
    h = jnp.dot(x, w1_ref[...], preferred_element_type=jnp.float32)
    h = h + b1e_ref[...][None, :]
    h = h * aff[0:1, :hid] + aff[1:2, :hid]
    h = jnp.maximum(h, 0.0)
    z = jnp.dot(h, w2_ref[...], preferred_element_type=jnp.float32)
    z = z + b2e_ref[...][None, :]

    zz = jnp.sum(z * z, axis=1)
    zcm2 = lax.dot_general(cb_ref[...] * -2.0, z,
                           (((1,), (1,)), ((), ())),
                           preferred_element_type=jnp.float32)
    d = (cc_scr[:, 0:1] + zz) + zcm2

    k = d.shape[0]
    mind = jnp.min(d, axis=0)
    ismin = d == mind[None, :]
    rowid = lax.broadcasted_iota(jnp.int32, (k, b), 0)
    idx = jnp.min(jnp.where(ismin, rowid, k), axis=0)
    topics_ref[...] = idx

    s_ref[...] += lax.dot_general(ismin.astype(jnp.bfloat16),
                                  x.astype(jnp.bfloat16),
                                  (((1,), (0,)), ((), ())),
                                  preferred_element_type=jnp.float32)
    rows = lax.broadcasted_iota(jnp.int32, (8, 128), 0)
    cols = lax.broadcasted_iota(jnp.int32, (8, 128), 1)
    st_ref[...] += jnp.where((rows == 0) & (cols == 0), jnp.sum(mind), 0.0)


def _hist_body(topics_hbm, out_hbm, idx_v, acc_v, red_v):
    nc = 2
    wid = lax.axis_index("s") * nc + lax.axis_index("c")
    shard = 2048
    kk = 1024
    base = wid * shard
    pltpu.sync_copy(topics_hbm.at[pl.ds(base, shard)], idx_v)

    zeros16 = jnp.zeros((16,), jnp.float32)

    def zbody(m, _):
        for u in range(8):
            acc_v[pl.ds(m * 128 + u * 16, 16)] = zeros16
        return _
    lax.fori_loop(0, (16 * kk) // 128, zbody, None)

    lane = lax.broadcasted_iota(jnp.int32, (16,), 0)
    bank = lane * kk
    ones16 = jnp.ones((16,), jnp.float32)

    def sbody(j, _):
        for u in range(4):
            t = idx_v[pl.ds(j * 64 + u * 16, 16)]
            plsc.addupdate_scatter(acc_v, [bank + t], ones16)
        return _
    lax.fori_loop(0, shard // 64, sbody, None)

    def rbody(i, _):
        s = zeros16
        for j in range(16):
            s = s + acc_v[pl.ds(j * kk + i * 16, 16)]
        red_v[pl.ds(i * 16, 16)] = s
        return _
    lax.fori_loop(0, kk // 16, rbody, None)

    pltpu.sync_copy(red_v, out_hbm.at[wid])


def _histogram_sc(topics, n, k):
    mesh = plsc.VectorSubcoreMesh(core_axis_name="c", subcore_axis_name="s")
    return pl.kernel(
        _hist_body,
        mesh=mesh,
        out_type=jax.ShapeDtypeStruct((32, k), jnp.float32),
        scratch_types=[pltpu.VMEM((n // 32,), jnp.int32),
                       pltpu.VMEM((16 * k,), jnp.float32),
                       pltpu.VMEM((k,), jnp.float32)],
        compiler_params=pltpu.CompilerParams(needs_layout_passes=False),
    )(topics)


def _combine_kernel(cb_ref, w1d_ref, w2d_ref, b1d_ref, g1d_ref, be1d_ref,
                    b2d_ref, cnt_ref, s_ref, st_ref, out_ref, *, n_rows):
    cb = cb_ref[...]
    hd = jnp.dot(cb, w1d_ref[...], preferred_element_type=jnp.float32)
    hd = hd + b1d_ref[...][None, :]
    counts = jnp.sum(cnt_ref[...], axis=0, keepdims=True)
    inv_n = 1.0 / n_rows
    mu = jnp.dot(counts, hd, preferred_element_type=jnp.float32) * inv_n
    ex2 = jnp.dot(counts, hd * hd, preferred_element_type=jnp.float32) * inv_n
    var = ex2 - mu * mu
    scale = g1d_ref[...][None, :] / jnp.sqrt(var + _BN_EPS)
    shift = be1d_ref[...][None, :] - mu * scale
    a = jnp.maximum(hd * scale + shift, 0.0)
    dec = jnp.dot(a, w2d_ref[...], preferred_element_type=jnp.float32)
    dec = dec + b2d_ref[...][None, :]

    d2 = jnp.sum(dec * dec, axis=1)
    cterm = jnp.sum(counts[0, :] * d2)
    cross = jnp.sum(dec * s_ref[...])

    r1 = lax.broadcasted_iota(jnp.int32, (8, 128), 0)
    c1 = lax.broadcasted_iota(jnp.int32, (8, 128), 1)
    stats = st_ref[...]
    sumx2 = jnp.sum(jnp.where((r1 == 2) & (c1 == 0), stats, 0.0))
    zloss = jnp.sum(jnp.where((r1 == 0) & (c1 == 0), stats, 0.0))

    recon = sumx2 + cterm - 2.0 * cross
    loss = 2.0 * zloss + jnp.sqrt(recon)
    out_ref[...] = jnp.full((8, 128), loss, dtype=jnp.float32)


def kernel(X, W1e, b1e, g1e, be1e, W2e, b2e, codebook,
           W1d, b1d, g1d, be1d, W2d, b2d):
    N, d_in = X.shape
    hid = W1e.shape[1]
    code = W2e.shape[1]
    K = codebook.shape[0]
    BN = 4096
    grid = N // BN

    stats1 = pl.pallas_call(
        _stats_kernel,
        grid=(grid,),
        in_specs=[pl.BlockSpec((BN, d_in), lambda i: (i, 0)),
                  pl.BlockSpec((d_in, hid), lambda i: (0, 0))],
        out_specs=pl.BlockSpec((8, 128), lambda i: (0, 0)),
        out_shape=jax.ShapeDtypeStruct((8, 128), jnp.float32),
    )(X, W1e)

    topics, S, st2 = pl.pallas_call(
        functools.partial(_main_kernel, n_rows=float(N)),
        grid=(grid,),
        in_specs=[pl.BlockSpec((BN, d_in), lambda i: (i, 0)),
                  pl.BlockSpec((d_in, hid), lambda i: (0, 0)),
                  pl.BlockSpec((hid, code), lambda i: (0, 0)),
                  pl.BlockSpec((K, code), lambda i: (0, 0)),
                  pl.BlockSpec((hid,), lambda i: (0,)),
                  pl.BlockSpec((hid,), lambda i: (0,)),
                  pl.BlockSpec((hid,), lambda i: (0,)),
                  pl.BlockSpec((code,), lambda i: (0,)),
                  pl.BlockSpec((8, 128), lambda i: (0, 0))],
        out_specs=[pl.BlockSpec((BN,), lambda i: (i,)),
                   pl.BlockSpec((K, d_in), lambda i: (0, 0)),
                   pl.BlockSpec((8, 128), lambda i: (0, 0))],
        out_shape=[jax.ShapeDtypeStruct((N,), jnp.int32),
                   jax.ShapeDtypeStruct((K, d_in), jnp.float32),
                   jax.ShapeDtypeStruct((8, 128), jnp.float32)],
        scratch_shapes=[pltpu.VMEM((8, 128), jnp.float32),
                        pltpu.VMEM((K, 8), jnp.float32)],
    )(X, W1e, W2e, codebook, b1e, g1e, be1e, b2e, stats1)

    counts_part = _histogram_sc(topics, N, K)

    loss_tile = pl.pallas_call(
        functools.partial(_combine_kernel, n_rows=float(N)),
        grid=(1,),
        in_specs=[pl.BlockSpec((K, code), lambda i: (0, 0)),
                  pl.BlockSpec((code, hid), lambda i: (0, 0)),
                  pl.BlockSpec((hid, d_in), lambda i: (0, 0)),
                  pl.BlockSpec((hid,), lambda i: (0,)),
                  pl.BlockSpec((hid,), lambda i: (0,)),
                  pl.BlockSpec((hid,), lambda i: (0,)),
                  pl.BlockSpec((d_in,), lambda i: (0,)),
                  pl.BlockSpec((32, K), lambda i: (0, 0)),
                  pl.BlockSpec((K, d_in), lambda i: (0, 0)),
                  pl.BlockSpec((8, 128), lambda i: (0, 0))],
        out_specs=pl.BlockSpec((8, 128), lambda i: (0, 0)),
        out_shape=jax.ShapeDtypeStruct((8, 128), jnp.float32),
    )(codebook, W1d, W2d, b1d, g1d, be1d, b2d, counts_part, S, st2)

    return (loss_tile[0, 0], topics)
